# baseline (speedup 1.0000x reference)
import functools

import numpy as np

import jax
import jax.numpy as jnp
from jax import lax
from jax.experimental import pallas as pl
from jax.experimental.pallas import tpu as pltpu
from jax.experimental.pallas import tpu_sc as plsc

_N = 10000
_G = 64
_NC = 2
_NS = 16
_NW = _NC * _NS
_CHUNK = 128
_NCH = 80
_EPAD = _NW * _NCH * _CHUNK
_ACC = 10112
_NPH = 5
_PCH = _NCH // _NPH

_PAD = _EPAD - 320000
_PAD_SRC = np.arange(_PAD, dtype=np.int32) % _N
_PAD_DST = _N + np.arange(_PAD, dtype=np.int32) % (_ACC - _N)


def _sc_edge_aggregate(y, src_t, dst_t, zeros):
  dout = y.shape[1]
  nbuf = 2 if dout == 128 else 4
  mesh = plsc.VectorSubcoreMesh(core_axis_name="c", subcore_axis_name="s")

  @functools.partial(
      pl.kernel,
      out_type=jax.ShapeDtypeStruct((_NC * _ACC, dout), jnp.float32),
      mesh=mesh,
      scratch_types=[
          pltpu.VMEM((_PCH, _CHUNK), jnp.int32),
          pltpu.VMEM((_PCH, _CHUNK), jnp.int32),
          pltpu.VMEM((_PCH, _CHUNK), jnp.int32),
          pltpu.VMEM((_PCH, _CHUNK), jnp.int32),
          [pltpu.VMEM((_CHUNK, dout), jnp.float32)] * nbuf,
          pltpu.VMEM_SHARED((_ACC, dout), jnp.float32),
          pltpu.SemaphoreType.DMA,
          pltpu.SemaphoreType.DMA,
          [pltpu.SemaphoreType.DMA] * nbuf,
          [pltpu.SemaphoreType.DMA] * nbuf,
      ],
      compiler_params=pltpu.CompilerParams(use_tc_tiling_on_sc=False),
  )
  def k(y_hbm, src_hbm, dst_hbm, z_hbm, out_hbm, sb0, db0, sb1, db1, gbufs,
        acc, isem0, isem1, gsems, ssems):
    c = lax.axis_index("c")
    s = lax.axis_index("s")
    wid = s * _NC + c

    def prow(p):
      return pl.ds((wid * _NPH + p) * _PCH, _PCH)

    pltpu.async_copy(src_hbm.at[prow(0)], sb0, isem0)
    pltpu.async_copy(dst_hbm.at[prow(0)], db0, isem0)
    pltpu.async_copy(src_hbm.at[prow(1)], sb1, isem1)
    pltpu.async_copy(dst_hbm.at[prow(1)], db1, isem1)
    zr = _ACC // _NS
    pltpu.sync_copy(z_hbm.at[pl.ds(s * zr, zr)], acc.at[pl.ds(s * zr, zr)])
    plsc.subcore_barrier()

    for p in range(_NPH):
      sb, db, isem = (sb0, db0, isem0) if p % 2 == 0 else (sb1, db1, isem1)
      pltpu.make_async_copy(src_hbm.at[prow(p)], sb, isem).wait()
      pltpu.make_async_copy(dst_hbm.at[prow(p)], db, isem).wait()

      def gissue(j, b, sb=sb):
        pltpu.async_copy(y_hbm.at[sb.at[j]], gbufs[b], gsems[b])

      def gwait(j, b, sb=sb):
        pltpu.make_async_copy(y_hbm.at[sb.at[j]], gbufs[b], gsems[b]).wait()

      def sissue(j, b, db=db):
        pltpu.async_copy(gbufs[b], acc.at[db.at[j]], ssems[b], add=True)

      def swait(b, db=db):
        pltpu.make_async_copy(gbufs[b], acc.at[db.at[0]], ssems[b]).wait()

      if nbuf == 2:
        gissue(0, 0)
        gissue(1, 1)

        def step(i, carry, sb=sb, db=db):
          for b in range(2):
            j = i * 2 + b
            gwait(j, b, sb)
            pltpu.sync_copy(gbufs[b], acc.at[db.at[j]], add=True)

            @pl.when(j + 2 < _PCH)
            def _():
              gissue(j + 2, b, sb)
          return carry

        lax.fori_loop(0, _PCH // 2, step, 0)
      else:
        gissue(0, 0)
        gissue(1, 1)
        for j in (0, 1):
          gwait(j, j)
          sissue(j, j)
          gissue(j + 2, j + 2)

        def step(i, carry, sb=sb, db=db):
          for u in range(4):
            j = i * 4 + 2 + u
            b = (2 + u) % 4
            gwait(j, b, sb)
            sissue(j, b, db)
            bn = u
            swait(bn, db)
            gissue(j + 2, bn, sb)
          return carry

        lax.fori_loop(0, (_PCH - 4) // 4, step, 0)
        for j in (_PCH - 2, _PCH - 1):
          b = j % 4
          gwait(j, b)
          sissue(j, b)
          swait((j + 2) % 4)
        swait((_PCH - 2) % 4)
        swait((_PCH - 1) % 4)

      if p + 2 < _NPH:
        pltpu.async_copy(src_hbm.at[prow(p + 2)], sb, isem)
        pltpu.async_copy(dst_hbm.at[prow(p + 2)], db, isem)

    plsc.subcore_barrier()
    orow = _ACC // _NS
    pltpu.sync_copy(acc.at[pl.ds(s * orow, orow)],
                    out_hbm.at[pl.ds(c * _ACC + s * orow, orow)])

  return k(y, src_t, dst_t, zeros)


def _tc_first(x, w_rel, w_root):
  dout = w_rel.shape[1]

  def body(x_r, wr_r, wo_r, y_r, r_r):
    xv = x_r[...]
    y_r[...] = jnp.dot(xv, wr_r[...], preferred_element_type=jnp.float32)
    r_r[...] = jnp.dot(xv, wo_r[...], preferred_element_type=jnp.float32)

  return pl.pallas_call(
      body,
      out_shape=(jax.ShapeDtypeStruct((_N, dout), jnp.float32),
                 jax.ShapeDtypeStruct((_N, dout), jnp.float32)),
  )(x, w_rel, w_root)


def _tc_mid(P, r, b2d, w_rel, w_root):
  dout = w_rel.shape[1]

  def body(p_r, r_r, b_r, wr_r, wo_r, y_r, q_r):
    h = p_r[:_N, :] + p_r[_ACC:_ACC + _N, :] + b_r[...] + r_r[...]
    h = jnp.maximum(h, 0.0)
    y_r[...] = jnp.dot(h, wr_r[...], preferred_element_type=jnp.float32)
    q_r[...] = jnp.dot(h, wo_r[...], preferred_element_type=jnp.float32)

  return pl.pallas_call(
      body,
      out_shape=(jax.ShapeDtypeStruct((_N, dout), jnp.float32),
                 jax.ShapeDtypeStruct((_N, dout), jnp.float32)),
  )(P, r, b2d, w_rel, w_root)


def _tc_head(P, r, b2d, batch2d, fc_w, fc_b2d):
  def body(p_r, r_r, b_r, bat_r, w_r, c_r, o_r):
    h = p_r[:_N, :] + p_r[_ACC:_ACC + _N, :] + b_r[...] + r_r[...]
    gid = lax.broadcasted_iota(jnp.int32, (_G, 1), 0)
    onehot = (bat_r[...] == gid).astype(jnp.float32)
    sums = jnp.dot(onehot, h, preferred_element_type=jnp.float32)
    cnt = jnp.sum(onehot, axis=1, keepdims=True)
    pooled = sums / jnp.maximum(cnt, 1.0)
    o_r[...] = jnp.dot(pooled, w_r[...],
                       preferred_element_type=jnp.float32) + c_r[...]

  return pl.pallas_call(
      body,
      out_shape=jax.ShapeDtypeStruct((_G, 1), jnp.float32),
  )(P, r, b2d, batch2d, fc_w, fc_b2d)


def kernel(x, edge_index, batch, W1_rel, b1_rel, W1_root, W2_rel, b2_rel,
           W2_root, W3_rel, b3_rel, W3_root, W4_rel, b4_rel, W4_root,
           fc_W, fc_b):
  src_t = jnp.concatenate(
      [edge_index[0], _PAD_SRC]).reshape(_NW * _NPH * _PCH, _CHUNK)
  dst_t = jnp.concatenate(
      [edge_index[1], _PAD_DST]).reshape(_NW * _NPH * _PCH, _CHUNK)
  batch2d = batch.reshape(1, _N)

  def zconst(dout):
    z = np.zeros((_ACC, dout), np.float32)
    z[_N, 0] = 1.0
    return z

  y, r = _tc_first(x, W1_rel, W1_root)
  P = _sc_edge_aggregate(y, src_t, dst_t, zconst(W1_rel.shape[1]))
  y, r = _tc_mid(P, r, b1_rel.reshape(1, -1), W2_rel, W2_root)
  P = _sc_edge_aggregate(y, src_t, dst_t, zconst(W2_rel.shape[1]))
  y, r = _tc_mid(P, r, b2_rel.reshape(1, -1), W3_rel, W3_root)
  P = _sc_edge_aggregate(y, src_t, dst_t, zconst(W3_rel.shape[1]))
  y, r = _tc_mid(P, r, b3_rel.reshape(1, -1), W4_rel, W4_root)
  P = _sc_edge_aggregate(y, src_t, dst_t, zconst(W4_rel.shape[1]))
  return _tc_head(P, r, b4_rel.reshape(1, -1), batch2d, fc_W,
                  fc_b.reshape(1, 1))


# speedup vs baseline: 1.0677x; 1.0677x over previous
import functools

import numpy as np

import jax
import jax.numpy as jnp
from jax import lax
from jax.experimental import pallas as pl
from jax.experimental.pallas import tpu as pltpu
from jax.experimental.pallas import tpu_sc as plsc

_N = 10000
_G = 64
_NC = 2
_NS = 16
_NW = _NC * _NS
_CHUNK = 128
_NCH = 80
_EPAD = _NW * _NCH * _CHUNK
_ACC = 10112
_NPH = 5
_PCH = _NCH // _NPH

_PAD = _EPAD - 320000
_PAD_SRC = np.arange(_PAD, dtype=np.int32) % _N
_PAD_DST = _N + np.arange(_PAD, dtype=np.int32) % (_ACC - _N)


def _sc_edge_aggregate(y, src_t, dst_t, zeros):
  dout = y.shape[1]
  nbuf = 2 if dout == 128 else 4
  out_shape = ((_NC * _ACC, dout) if dout == 128 else (_ACC, 128))
  mesh = plsc.VectorSubcoreMesh(core_axis_name="c", subcore_axis_name="s")

  @functools.partial(
      pl.kernel,
      out_type=jax.ShapeDtypeStruct(out_shape, jnp.float32),
      mesh=mesh,
      scratch_types=[
          pltpu.VMEM((_PCH, _CHUNK), jnp.int32),
          pltpu.VMEM((_PCH, _CHUNK), jnp.int32),
          pltpu.VMEM((_PCH, _CHUNK), jnp.int32),
          pltpu.VMEM((_PCH, _CHUNK), jnp.int32),
          [pltpu.VMEM((_CHUNK, dout), jnp.float32)] * nbuf,
          pltpu.VMEM_SHARED((_ACC, dout), jnp.float32),
          pltpu.SemaphoreType.DMA,
          pltpu.SemaphoreType.DMA,
          [pltpu.SemaphoreType.DMA] * nbuf,
          [pltpu.SemaphoreType.DMA] * nbuf,
      ],
      compiler_params=pltpu.CompilerParams(use_tc_tiling_on_sc=False),
  )
  def k(y_hbm, src_hbm, dst_hbm, z_hbm, out_hbm, sb0, db0, sb1, db1, gbufs,
        acc, isem0, isem1, gsems, ssems):
    c = lax.axis_index("c")
    s = lax.axis_index("s")
    wid = s * _NC + c

    def prow(p):
      return pl.ds((wid * _NPH + p) * _PCH, _PCH)

    pltpu.async_copy(src_hbm.at[prow(0)], sb0, isem0)
    pltpu.async_copy(dst_hbm.at[prow(0)], db0, isem0)
    pltpu.async_copy(src_hbm.at[prow(1)], sb1, isem1)
    pltpu.async_copy(dst_hbm.at[prow(1)], db1, isem1)
    zr = _ACC // _NS
    pltpu.sync_copy(z_hbm.at[pl.ds(s * zr, zr)], acc.at[pl.ds(s * zr, zr)])
    plsc.subcore_barrier()

    for p in range(_NPH):
      sb, db, isem = (sb0, db0, isem0) if p % 2 == 0 else (sb1, db1, isem1)
      pltpu.make_async_copy(src_hbm.at[prow(p)], sb, isem).wait()
      pltpu.make_async_copy(dst_hbm.at[prow(p)], db, isem).wait()

      def gissue(j, b, sb=sb):
        pltpu.async_copy(y_hbm.at[sb.at[j]], gbufs[b], gsems[b])

      def gwait(j, b, sb=sb):
        pltpu.make_async_copy(y_hbm.at[sb.at[j]], gbufs[b], gsems[b]).wait()

      def sissue(j, b, db=db):
        pltpu.async_copy(gbufs[b], acc.at[db.at[j]], ssems[b], add=True)

      def swait(b, db=db):
        pltpu.make_async_copy(gbufs[b], acc.at[db.at[0]], ssems[b]).wait()

      if nbuf == 2:
        gissue(0, 0)
        gissue(1, 1)

        def step(i, carry, sb=sb, db=db):
          for b in range(2):
            j = i * 2 + b
            gwait(j, b, sb)
            pltpu.sync_copy(gbufs[b], acc.at[db.at[j]], add=True)

            @pl.when(j + 2 < _PCH)
            def _():
              gissue(j + 2, b, sb)
          return carry

        lax.fori_loop(0, _PCH // 2, step, 0)
      else:
        gissue(0, 0)
        gissue(1, 1)
        for j in (0, 1):
          gwait(j, j)
          sissue(j, j)
          gissue(j + 2, j + 2)

        def step(i, carry, sb=sb, db=db):
          for u in range(4):
            j = i * 4 + 2 + u
            b = (2 + u) % 4
            gwait(j, b, sb)
            sissue(j, b, db)
            bn = u
            swait(bn, db)
            gissue(j + 2, bn, sb)
          return carry

        lax.fori_loop(0, (_PCH - 4) // 4, step, 0)
        for j in (_PCH - 2, _PCH - 1):
          b = j % 4
          gwait(j, b)
          sissue(j, b)
          swait((j + 2) % 4)
        swait((_PCH - 2) % 4)
        swait((_PCH - 1) % 4)

      if p + 2 < _NPH:
        pltpu.async_copy(src_hbm.at[prow(p + 2)], sb, isem)
        pltpu.async_copy(dst_hbm.at[prow(p + 2)], db, isem)

    plsc.subcore_barrier()
    orow = _ACC // _NS
    if dout == 128:
      pltpu.sync_copy(acc.at[pl.ds(s * orow, orow)],
                      out_hbm.at[pl.ds(c * _ACC + s * orow, orow)])
    else:
      pltpu.sync_copy(acc.at[pl.ds(s * orow, orow)],
                      out_hbm.at[pl.ds(s * orow, orow),
                                 pl.ds(c * dout, dout)])

  return k(y, src_t, dst_t, zeros)


def _tc_first(x, w_rel, w_root):
  dout = w_rel.shape[1]

  def body(x_r, wr_r, wo_r, y_r, r_r):
    xv = x_r[...]
    y_r[...] = jnp.dot(xv, wr_r[...], preferred_element_type=jnp.float32)
    r_r[...] = jnp.dot(xv, wo_r[...], preferred_element_type=jnp.float32)

  return pl.pallas_call(
      body,
      out_shape=(jax.ShapeDtypeStruct((_N, dout), jnp.float32),
                 jax.ShapeDtypeStruct((_N, dout), jnp.float32)),
  )(x, w_rel, w_root)


def _tc_mid(P, r, b2d, w_rel, w_root):
  dout = w_rel.shape[1]
  dprev = r.shape[1]

  def body(p_r, r_r, b_r, wr_r, wo_r, y_r, q_r):
    if P.shape[0] == _NC * _ACC:
      psum = p_r[:_N, :] + p_r[_ACC:_ACC + _N, :]
    else:
      psum = p_r[:_N, :dprev] + p_r[:_N, dprev:2 * dprev]
    h = psum + b_r[...] + r_r[...]
    h = jnp.maximum(h, 0.0)
    y_r[...] = jnp.dot(h, wr_r[...], preferred_element_type=jnp.float32)
    q_r[...] = jnp.dot(h, wo_r[...], preferred_element_type=jnp.float32)

  return pl.pallas_call(
      body,
      out_shape=(jax.ShapeDtypeStruct((_N, dout), jnp.float32),
                 jax.ShapeDtypeStruct((_N, dout), jnp.float32)),
  )(P, r, b2d, w_rel, w_root)


def _tc_head(P, r, b2d, batch2d, fc_w, fc_b2d):
  dprev = r.shape[1]

  def body(p_r, r_r, b_r, bat_r, w_r, c_r, o_r):
    psum = p_r[:_N, :dprev] + p_r[:_N, dprev:2 * dprev]
    h = psum + b_r[...] + r_r[...]
    gid = lax.broadcasted_iota(jnp.int32, (_G, 1), 0)
    onehot = (bat_r[...] == gid).astype(jnp.float32)
    sums = jnp.dot(onehot, h, preferred_element_type=jnp.float32)
    cnt = jnp.sum(onehot, axis=1, keepdims=True)
    pooled = sums / jnp.maximum(cnt, 1.0)
    o_r[...] = jnp.dot(pooled, w_r[...],
                       preferred_element_type=jnp.float32) + c_r[...]

  return pl.pallas_call(
      body,
      out_shape=jax.ShapeDtypeStruct((_G, 1), jnp.float32),
  )(P, r, b2d, batch2d, fc_w, fc_b2d)


def kernel(x, edge_index, batch, W1_rel, b1_rel, W1_root, W2_rel, b2_rel,
           W2_root, W3_rel, b3_rel, W3_root, W4_rel, b4_rel, W4_root,
           fc_W, fc_b):
  src_t = jnp.concatenate(
      [edge_index[0], _PAD_SRC]).reshape(_NW * _NPH * _PCH, _CHUNK)
  dst_t = jnp.concatenate(
      [edge_index[1], _PAD_DST]).reshape(_NW * _NPH * _PCH, _CHUNK)
  batch2d = batch.reshape(1, _N)

  def zconst(dout):
    z = np.zeros((_ACC, dout), np.float32)
    z[_N, 0] = 1.0
    return z

  y, r = _tc_first(x, W1_rel, W1_root)
  P = _sc_edge_aggregate(y, src_t, dst_t, zconst(W1_rel.shape[1]))
  y, r = _tc_mid(P, r, b1_rel.reshape(1, -1), W2_rel, W2_root)
  P = _sc_edge_aggregate(y, src_t, dst_t, zconst(W2_rel.shape[1]))
  y, r = _tc_mid(P, r, b2_rel.reshape(1, -1), W3_rel, W3_root)
  P = _sc_edge_aggregate(y, src_t, dst_t, zconst(W3_rel.shape[1]))
  y, r = _tc_mid(P, r, b3_rel.reshape(1, -1), W4_rel, W4_root)
  P = _sc_edge_aggregate(y, src_t, dst_t, zconst(W4_rel.shape[1]))
  return _tc_head(P, r, b4_rel.reshape(1, -1), batch2d, fc_W,
                  fc_b.reshape(1, 1))
